# baseline (speedup 1.0000x reference)
import jax
import jax.numpy as jnp
from jax.experimental import pallas as pl
from jax.experimental.pallas import tpu as pltpu


def _fused_kernel(x_ref, wgc_ref, fadj_ref, bgc_ref, wfc_ref, bfc_ref,
                  out_ref, sup_ref):
    @pl.when(pl.program_id(0) == 0)
    def _():
        sup_ref[...] = jnp.dot(
            x_ref[...].astype(jnp.bfloat16),
            wgc_ref[...].astype(jnp.bfloat16),
            preferred_element_type=jnp.float32).astype(jnp.bfloat16)

    a = fadj_ref[...].astype(jnp.bfloat16)
    h = jnp.dot(a, sup_ref[...],
                preferred_element_type=jnp.float32) + bgc_ref[...]
    h = jnp.where(h > 0, h, jnp.exp(jnp.minimum(h, 0.0)) - 1.0)
    out_ref[...] = (jnp.dot(h, wfc_ref[...],
                            preferred_element_type=jnp.float32)
                    + bfc_ref[...])


@jax.jit
def kernel(input, fadj, W_gc, b_gc, W_fc, b_fc):
    n, n_in = input.shape
    nfea = W_gc.shape[1]
    n_class = W_fc.shape[1]

    bm = 200
    out = pl.pallas_call(
        _fused_kernel,
        grid=(n // bm,),
        in_specs=[
            pl.BlockSpec((n, n_in), lambda i: (0, 0)),
            pl.BlockSpec((n_in, nfea), lambda i: (0, 0)),
            pl.BlockSpec((bm, n), lambda i: (i, 0)),
            pl.BlockSpec((1, nfea), lambda i: (0, 0)),
            pl.BlockSpec((nfea, n_class), lambda i: (0, 0)),
            pl.BlockSpec((1, n_class), lambda i: (0, 0)),
        ],
        out_specs=pl.BlockSpec((bm, n_class), lambda i: (i, 0)),
        out_shape=jax.ShapeDtypeStruct((n, n_class), jnp.float32),
        scratch_shapes=[pltpu.VMEM((n, nfea), jnp.bfloat16)],
    )(input, W_gc, fadj, b_gc.reshape(1, nfea), W_fc,
      b_fc.reshape(1, n_class))
    return out


# speedup vs baseline: 1.0136x; 1.0136x over previous
import functools

import jax
import jax.numpy as jnp
from jax.experimental import pallas as pl
from jax.experimental.pallas import tpu as pltpu

_NBUF = 3


def _fused_kernel(bm, x_ref, wgc_ref, bgc_ref, wfc_ref, bfc_ref, fadj_hbm,
                  out_ref, sup_ref, buf_ref, sem):
    i = pl.program_id(0)
    nsteps = pl.num_programs(0)

    @pl.when(i == 0)
    def _():
        sup_ref[...] = jnp.dot(
            x_ref[...].astype(jnp.bfloat16),
            wgc_ref[...].astype(jnp.bfloat16),
            preferred_element_type=jnp.float32).astype(jnp.bfloat16)
        for s in range(_NBUF):
            pltpu.make_async_copy(
                fadj_hbm.at[pl.ds(s * bm, bm), :],
                buf_ref.at[s], sem.at[s]).start()

    nxt = i + _NBUF - 1

    @pl.when((i > 0) & (nxt < nsteps))
    def _():
        slot = jax.lax.rem(nxt, _NBUF)
        pltpu.make_async_copy(
            fadj_hbm.at[pl.ds(nxt * bm, bm), :],
            buf_ref.at[slot], sem.at[slot]).start()

    slot_i = jax.lax.rem(i, _NBUF)
    pltpu.make_async_copy(
        fadj_hbm.at[pl.ds(i * bm, bm), :],
        buf_ref.at[slot_i], sem.at[slot_i]).wait()

    a = buf_ref[slot_i].astype(jnp.bfloat16)
    h = jnp.dot(a, sup_ref[...],
                preferred_element_type=jnp.float32) + bgc_ref[...]
    h = jnp.where(h > 0, h, jnp.exp(jnp.minimum(h, 0.0)) - 1.0)
    out_ref[...] = (jnp.dot(h, wfc_ref[...],
                            preferred_element_type=jnp.float32)
                    + bfc_ref[...])


@jax.jit
def kernel(input, fadj, W_gc, b_gc, W_fc, b_fc):
    n, n_in = input.shape
    nfea = W_gc.shape[1]
    n_class = W_fc.shape[1]

    bm = 400
    out = pl.pallas_call(
        functools.partial(_fused_kernel, bm),
        grid=(n // bm,),
        in_specs=[
            pl.BlockSpec((n, n_in), lambda i: (0, 0)),
            pl.BlockSpec((n_in, nfea), lambda i: (0, 0)),
            pl.BlockSpec((1, nfea), lambda i: (0, 0)),
            pl.BlockSpec((nfea, n_class), lambda i: (0, 0)),
            pl.BlockSpec((1, n_class), lambda i: (0, 0)),
            pl.BlockSpec(memory_space=pltpu.MemorySpace.HBM),
        ],
        out_specs=pl.BlockSpec((bm, n_class), lambda i: (i, 0)),
        out_shape=jax.ShapeDtypeStruct((n, n_class), jnp.float32),
        compiler_params=pltpu.CompilerParams(vmem_limit_bytes=64 * 1024 * 1024),
        scratch_shapes=[
            pltpu.VMEM((n, nfea), jnp.bfloat16),
            pltpu.VMEM((_NBUF, bm, n), jnp.float32),
            pltpu.SemaphoreType.DMA((_NBUF,)),
        ],
    )(input, W_gc, b_gc.reshape(1, nfea), W_fc, b_fc.reshape(1, n_class),
      fadj)
    return out
